# Initial kernel scaffold; baseline (speedup 1.0000x reference)
import jax
import jax.numpy as jnp
from jax.experimental import pallas as pl


def kernel(x, edge_index, W1, attn1, W2, attn2):
    raise NotImplementedError("write your pallas kernel here")


# speedup vs baseline: 1.0102x; 1.0102x over previous
import functools

import jax
import jax.numpy as jnp
from jax import lax
from jax.experimental import pallas as pl
from jax.experimental.pallas import tpu as pltpu

N = 10000
E = 160000
H = 4
NEG_SLOPE = 0.2

_M_TILE = 400


def _mm_body(h_ref, w_ref, attn_ref, f_ref, g_ref, gmax_ref):
    i = pl.program_id(0)
    f = jnp.dot(h_ref[...], w_ref[...], preferred_element_type=jnp.float32,
                precision=lax.Precision.HIGHEST)
    f_ref[...] = f
    a = attn_ref[...].reshape(1, -1)
    af = f * a
    lr = jnp.maximum(af, NEG_SLOPE * af)
    d = f.shape[1] // H
    g = lr.reshape(_M_TILE, H, d).sum(-1)
    g_ref[...] = g
    tile_max = jnp.max(g)

    @pl.when(i == 0)
    def _init():
        gmax_ref[0, 0] = tile_max

    @pl.when(i > 0)
    def _acc():
        gmax_ref[0, 0] = jnp.maximum(gmax_ref[0, 0], tile_max)


def _matmul_g(h, W, attn):
    n, k = h.shape
    hd = W.shape[1]
    grid = n // _M_TILE
    f, g, gmax = pl.pallas_call(
        _mm_body,
        grid=(grid,),
        in_specs=[
            pl.BlockSpec((_M_TILE, k), lambda i: (i, 0)),
            pl.BlockSpec((k, hd), lambda i: (0, 0)),
            pl.BlockSpec((H, hd // H), lambda i: (0, 0)),
        ],
        out_specs=[
            pl.BlockSpec((_M_TILE, hd), lambda i: (i, 0)),
            pl.BlockSpec((_M_TILE, H), lambda i: (i, 0)),
            pl.BlockSpec(memory_space=pltpu.SMEM),
        ],
        out_shape=[
            jax.ShapeDtypeStruct((n, hd), jnp.float32),
            jax.ShapeDtypeStruct((n, H), jnp.float32),
            jax.ShapeDtypeStruct((1, 1), jnp.float32),
        ],
    )(h, W, attn)
    return f, g, gmax


def _edge_layer(f, g, gmax, src, dst, h_dst, attn, residual):
    d = attn.shape[1]
    fs = f.reshape(N, H, d)
    z = fs[src] + fs[dst]
    lr = jnp.maximum(z, NEG_SLOPE * z)
    logits = (lr * attn[None]).sum(-1)
    m = g + gmax[0, 0]
    ex = jnp.exp(logits - m[dst])
    denom = jax.ops.segment_sum(ex, dst, num_segments=N)
    alpha = ex / jnp.maximum(denom[dst], 1e-9)
    rst = jax.ops.segment_sum(fs[src] * alpha[..., None], dst, num_segments=N)
    if residual:
        rst = rst + h_dst.reshape(N, H, d)
    return rst


def kernel(x, edge_index, W1, attn1, W2, attn2):
    src = edge_index[0]
    dst = edge_index[1]
    f1, g1, gmax1 = _matmul_g(x, W1, attn1)
    h = _edge_layer(f1, g1, gmax1, src, dst, x, attn1, residual=False)
    h = jax.nn.relu(h).reshape(N, H * 256)
    f2, g2, gmax2 = _matmul_g(h, W2, attn2)
    out = _edge_layer(f2, g2, gmax2, src, dst, h, attn2, residual=True)
    return out.mean(axis=1)


# speedup vs baseline: 2.0957x; 2.0744x over previous
import functools

import jax
import jax.numpy as jnp
from jax import lax
from jax.experimental import pallas as pl
from jax.experimental.pallas import tpu as pltpu
from jax.experimental.pallas import tpu_sc as plsc

N = 10000
E = 160000
H = 4
D = 256
HD = H * D
NEG_SLOPE = 0.2
NW = 32
EPAD = E + 32
RPPAD = 10008
VSPAD = 48

_M_TILE = 400


def _mm_body(h_ref, w_ref, attn_ref, f_ref, g_ref, gmax_ref):
    i = pl.program_id(0)
    f = jnp.dot(h_ref[...], w_ref[...], preferred_element_type=jnp.float32,
                precision=lax.Precision.HIGHEST)
    f_ref[...] = f
    a = attn_ref[...].reshape(1, -1)
    af = f * a
    lr = jnp.maximum(af, NEG_SLOPE * af)
    g = lr.reshape(_M_TILE, H, -1).sum(-1)
    g_ref[...] = g
    tile_max = jnp.max(g)

    @pl.when(i == 0)
    def _init():
        gmax_ref[0, 0] = tile_max

    @pl.when(i > 0)
    def _acc():
        gmax_ref[0, 0] = jnp.maximum(gmax_ref[0, 0], tile_max)


def _matmul_g(h, W, attn):
    n, k = h.shape
    hd = W.shape[1]
    f, g, gmax = pl.pallas_call(
        _mm_body,
        grid=(n // _M_TILE,),
        in_specs=[
            pl.BlockSpec((_M_TILE, k), lambda i: (i, 0)),
            pl.BlockSpec((k, hd), lambda i: (0, 0)),
            pl.BlockSpec((H, hd // H), lambda i: (0, 0)),
        ],
        out_specs=[
            pl.BlockSpec((_M_TILE, hd), lambda i: (i, 0)),
            pl.BlockSpec((_M_TILE, H), lambda i: (i, 0)),
            pl.BlockSpec(memory_space=pltpu.SMEM),
        ],
        out_shape=[
            jax.ShapeDtypeStruct((n, hd), jnp.float32),
            jax.ShapeDtypeStruct((n, H), jnp.float32),
            jax.ShapeDtypeStruct((1, 1), jnp.float32),
        ],
    )(h, W, attn)
    return f, g, gmax


def _make_edge_kernel(residual_mean):
    out_dim = D if residual_mean else HD
    mesh = plsc.VectorSubcoreMesh(core_axis_name="c", subcore_axis_name="s")
    scratch = [
        pltpu.VMEM((N * H,), jnp.float32),
        pltpu.VMEM((RPPAD,), jnp.int32),
        pltpu.VMEM((VSPAD,), jnp.int32),
        pltpu.VMEM((16,), jnp.float32),
        pltpu.VMEM((HD,), jnp.float32),
        pltpu.VMEM((HD,), jnp.float32),
        pltpu.VMEM((16, HD), jnp.float32),
        pltpu.VMEM((HD,), jnp.float32),
        pltpu.VMEM((out_dim,), jnp.float32),
        pltpu.VMEM((HD,), jnp.float32),
        pltpu.VMEM((16,), jnp.int32),
        pltpu.VMEM((32,), jnp.int32),
        pltpu.SemaphoreType.DMA,
    ]

    @functools.partial(
        pl.kernel,
        out_type=jax.ShapeDtypeStruct((N, out_dim), jnp.float32),
        mesh=mesh,
        scratch_types=scratch,
        compiler_params=pltpu.CompilerParams(needs_layout_passes=False),
    )
    def edge_kernel(f_hbm, g_hbm, gmax_hbm, srcs_hbm, rp_hbm, vs_hbm, attn_hbm,
                    hres_hbm, out_hbm, g_v, rp_v, vs_v, gmax_v, attn_v, fd_v,
                    fu_v, racc_v, stg_v, hrow_v, idx_v, sst_v, sem):
        cc = lax.axis_index("c")
        sc = lax.axis_index("s")
        wid = sc * 2 + cc
        pltpu.sync_copy(g_hbm, g_v)
        pltpu.sync_copy(rp_hbm, rp_v)
        pltpu.sync_copy(vs_hbm, vs_v)
        pltpu.sync_copy(gmax_hbm, gmax_v)
        pltpu.sync_copy(attn_hbm, attn_v)
        lanes = lax.iota(jnp.int32, 16)
        zero16 = jnp.zeros((16,), jnp.float32)
        gmaxv = gmax_v[...]

        def spl(x):
            return jnp.full((16,), x, jnp.int32)

        def ext(ref, i):
            return jnp.max(plsc.load_gather(ref, [spl(i)]))

        v0 = ext(vs_v, wid)
        v1 = ext(vs_v, wid + 1)

        def per_dst(v, carry):
            lo = ext(rp_v, v)
            hi = ext(rp_v, v + 1)
            pltpu.sync_copy(f_hbm.at[v], fd_v)

            def zr(i, c2):
                racc_v[pl.ds(i * 16, 16)] = zero16
                return c2

            lax.fori_loop(0, HD // 16, zr, 0)
            nch = (hi - lo + 15) // 16

            def per_chunk(ci, dcar):
                base = lo + ci * 16
                k = hi - base
                off8 = pl.multiple_of((base // 8) * 8, 8)
                pltpu.sync_copy(srcs_hbm.at[pl.ds(off8, 32)], sst_v)
                pos = (base - off8) + jnp.minimum(lanes, k - 1)
                idx_v[...] = plsc.load_gather(sst_v, [pos])
                pltpu.async_copy(f_hbm.at[idx_v], fu_v, sem).wait()
                mask = lanes < k
                newd = []
                exs = []
                for h in range(H):
                    def feat(d, car):
                        az_, aaz_ = car
                        spd = spl(h * D + d)
                        fu = plsc.load_gather(fu_v, [lanes, spd])
                        fdd = plsc.load_gather(fd_v, [spd])
                        ad = plsc.load_gather(attn_v, [spd])
                        z = fu + fdd
                        return az_ + ad * z, aaz_ + ad * jnp.abs(z)

                    az, aaz = lax.fori_loop(0, D, feat, (zero16, zero16))
                    logit = (0.5 * (1 + NEG_SLOPE)) * az + (0.5 * (1 - NEG_SLOPE)) * aaz
                    mh = plsc.load_gather(g_v, [spl(v * H + h)]) + gmaxv
                    ex = jnp.exp(logit - mh)
                    ex = jnp.where(mask, ex, 0.0)
                    exs.append(ex)
                    newd.append(dcar[h] + ex)
                for h in range(H):
                    exj = [jnp.full((16,), jnp.max(
                        jnp.where(lanes == j, exs[h], 0.0)))
                           for j in range(16)]

                    def agg(si, c3):
                        off = h * D + si * 16
                        r = racc_v[pl.ds(off, 16)]
                        for j in range(16):
                            r = r + exj[j] * fu_v[j, pl.ds(off, 16)]
                        racc_v[pl.ds(off, 16)] = r
                        return c3

                    lax.fori_loop(0, D // 16, agg, 0)
                return tuple(newd)

            dfin = lax.fori_loop(0, nch, per_chunk, (zero16,) * H)
            invs = [1.0 / jnp.maximum(jnp.full((16,), jnp.sum(dfin[h])), 1e-9)
                    for h in range(H)]
            if residual_mean:
                pltpu.sync_copy(hres_hbm.at[v], hrow_v)

                def fl(si, c4):
                    acc = zero16
                    for h in range(H):
                        off = h * D + si * 16
                        acc = acc + racc_v[pl.ds(off, 16)] * invs[h] \
                            + hrow_v[pl.ds(off, 16)]
                    stg_v[pl.ds(si * 16, 16)] = acc * (1.0 / H)
                    return c4
            else:
                def fl(si, c4):
                    for h in range(H):
                        off = h * D + si * 16
                        r = racc_v[pl.ds(off, 16)] * invs[h]
                        stg_v[pl.ds(off, 16)] = jnp.maximum(r, 0.0)
                    return c4

            lax.fori_loop(0, D // 16, fl, 0)
            pltpu.sync_copy(stg_v, out_hbm.at[v])
            return carry

        lax.fori_loop(v0, v1, per_dst, 0)

    return edge_kernel


def _edge_setup(src, dst):
    order = jnp.argsort(dst)
    src_s = src[order].astype(jnp.int32)
    dst_s = dst[order]
    rowptr = jnp.searchsorted(dst_s, jnp.arange(N + 1, dtype=jnp.int32)
                              ).astype(jnp.int32)
    targets = (jnp.arange(NW + 1, dtype=jnp.int32) * E) // NW
    vstart = jnp.searchsorted(rowptr, targets).astype(jnp.int32)
    vstart = jnp.minimum(vstart, N).at[NW].set(N).at[0].set(0)
    srcs_pad = jnp.pad(src_s, (0, EPAD - E))
    rowptr_pad = jnp.pad(rowptr, (0, RPPAD - (N + 1)))
    vstart_pad = jnp.pad(vstart, (0, VSPAD - (NW + 1)))
    return srcs_pad, rowptr_pad, vstart_pad


def kernel(x, edge_index, W1, attn1, W2, attn2):
    src = edge_index[0]
    dst = edge_index[1]
    srcs_pad, rowptr_pad, vstart_pad = _edge_setup(src, dst)

    f1, g1, gm1 = _matmul_g(x, W1, attn1)
    gm1b = jnp.broadcast_to(gm1.reshape(1), (16,))
    h = _make_edge_kernel(False)(
        f1, g1.reshape(-1), gm1b, srcs_pad, rowptr_pad, vstart_pad,
        attn1.reshape(-1), f1)

    f2, g2, gm2 = _matmul_g(h, W2, attn2)
    gm2b = jnp.broadcast_to(gm2.reshape(1), (16,))
    out = _make_edge_kernel(True)(
        f2, g2.reshape(-1), gm2b, srcs_pad, rowptr_pad, vstart_pad,
        attn2.reshape(-1), h)
    return out


# speedup vs baseline: 3.7577x; 1.7931x over previous
import functools

import jax
import jax.numpy as jnp
from jax import lax
from jax.experimental import pallas as pl
from jax.experimental.pallas import tpu as pltpu
from jax.experimental.pallas import tpu_sc as plsc

N = 10000
E = 160000
H = 4
D = 256
HD = H * D
NEG_SLOPE = 0.2
NW = 32
EPAD = E + 32
RPPAD = 10008
VSPAD = 48

_M_TILE = 400


def _mm_body(h_ref, w_ref, attn_ref, f_ref, g_ref, gmax_ref):
    i = pl.program_id(0)
    f = jnp.dot(h_ref[...], w_ref[...], preferred_element_type=jnp.float32,
                precision=lax.Precision.HIGHEST)
    f_ref[...] = f
    a = attn_ref[...].reshape(1, -1)
    af = f * a
    lr = jnp.maximum(af, NEG_SLOPE * af)
    g = lr.reshape(_M_TILE, H, -1).sum(-1)
    g_ref[...] = g
    tile_max = jnp.max(g)

    @pl.when(i == 0)
    def _init():
        gmax_ref[0, 0] = tile_max

    @pl.when(i > 0)
    def _acc():
        gmax_ref[0, 0] = jnp.maximum(gmax_ref[0, 0], tile_max)


def _matmul_g(h, W, attn):
    n, k = h.shape
    hd = W.shape[1]
    f, g, gmax = pl.pallas_call(
        _mm_body,
        grid=(n // _M_TILE,),
        in_specs=[
            pl.BlockSpec((_M_TILE, k), lambda i: (i, 0)),
            pl.BlockSpec((k, hd), lambda i: (0, 0)),
            pl.BlockSpec((H, hd // H), lambda i: (0, 0)),
        ],
        out_specs=[
            pl.BlockSpec((_M_TILE, hd), lambda i: (i, 0)),
            pl.BlockSpec((_M_TILE, H), lambda i: (i, 0)),
            pl.BlockSpec(memory_space=pltpu.SMEM),
        ],
        out_shape=[
            jax.ShapeDtypeStruct((n, hd), jnp.float32),
            jax.ShapeDtypeStruct((n, H), jnp.float32),
            jax.ShapeDtypeStruct((1, 1), jnp.float32),
        ],
    )(h, W, attn)
    return f, g, gmax


def _make_edge_kernel(residual_mean):
    out_dim = D if residual_mean else HD
    mesh = plsc.VectorSubcoreMesh(core_axis_name="c", subcore_axis_name="s")
    scratch = [
        pltpu.VMEM((N * H,), jnp.float32),
        pltpu.VMEM((RPPAD,), jnp.int32),
        pltpu.VMEM((VSPAD,), jnp.int32),
        pltpu.VMEM((16,), jnp.float32),
        pltpu.VMEM((HD,), jnp.float32),
        pltpu.VMEM((HD,), jnp.float32),
        pltpu.VMEM((16, HD), jnp.float32),
        pltpu.VMEM((HD,), jnp.float32),
        pltpu.VMEM((out_dim,), jnp.float32),
        pltpu.VMEM((HD,), jnp.float32),
        pltpu.VMEM((16,), jnp.int32),
        pltpu.VMEM((32,), jnp.int32),
        pltpu.SemaphoreType.DMA,
    ]

    @functools.partial(
        pl.kernel,
        out_type=jax.ShapeDtypeStruct((N, out_dim), jnp.float32),
        mesh=mesh,
        scratch_types=scratch,
        compiler_params=pltpu.CompilerParams(needs_layout_passes=False),
    )
    def edge_kernel(f_hbm, g_hbm, gmax_hbm, srcs_hbm, rp_hbm, vs_hbm, attn_hbm,
                    hres_hbm, out_hbm, g_v, rp_v, vs_v, gmax_v, attn_v, fd_v,
                    fu_v, racc_v, stg_v, hrow_v, idx_v, sst_v, sem):
        cc = lax.axis_index("c")
        sc = lax.axis_index("s")
        wid = sc * 2 + cc
        pltpu.sync_copy(g_hbm, g_v)
        pltpu.sync_copy(rp_hbm, rp_v)
        pltpu.sync_copy(vs_hbm, vs_v)
        pltpu.sync_copy(gmax_hbm, gmax_v)
        pltpu.sync_copy(attn_hbm, attn_v)
        lanes = lax.iota(jnp.int32, 16)
        zero16 = jnp.zeros((16,), jnp.float32)
        gmaxv = gmax_v[...]

        def spl(x):
            return jnp.full((16,), x, jnp.int32)

        def ext(ref, i):
            return jnp.max(plsc.load_gather(ref, [spl(i)]))

        v0 = ext(vs_v, wid)
        v1 = ext(vs_v, wid + 1)

        def per_dst(v, carry):
            lo = ext(rp_v, v)
            hi = ext(rp_v, v + 1)
            pltpu.sync_copy(f_hbm.at[v], fd_v)

            def zr(i, c2):
                racc_v[pl.ds(i * 16, 16)] = zero16
                return c2

            lax.fori_loop(0, HD // 16, zr, 0)
            nch = (hi - lo + 15) // 16

            def per_chunk(ci, dcar):
                base = lo + ci * 16
                k = hi - base
                off8 = pl.multiple_of((base // 8) * 8, 8)
                pltpu.sync_copy(srcs_hbm.at[pl.ds(off8, 32)], sst_v)
                pos = (base - off8) + jnp.minimum(lanes, k - 1)
                idx_v[...] = plsc.load_gather(sst_v, [pos])
                pltpu.async_copy(f_hbm.at[idx_v], fu_v, sem).wait()
                mask = lanes < k
                c6 = 0.5 * (1 + NEG_SLOPE)
                c4 = 0.5 * (1 - NEG_SLOPE)
                newd = []
                for h in range(H):
                    def lg(s16, accs_t):
                        dbase = h * D + s16 * 16
                        fdv = fd_v[pl.ds(dbase, 16)]
                        av = attn_v[pl.ds(dbase, 16)]
                        a6 = c6 * av
                        a4 = c4 * av
                        out = []
                        for j in range(16):
                            z = fu_v[j, pl.ds(dbase, 16)] + fdv
                            out.append(accs_t[j] + a6 * z + a4 * jnp.abs(z))
                        return tuple(out)

                    accs = lax.fori_loop(0, D // 16, lg, (zero16,) * 16)
                    logitv = zero16
                    for j in range(16):
                        logitv = jnp.where(lanes == j,
                                           jnp.full((16,), jnp.sum(accs[j])),
                                           logitv)
                    mh = plsc.load_gather(g_v, [spl(v * H + h)]) + gmaxv
                    ex = jnp.exp(logitv - mh)
                    ex = jnp.where(mask, ex, 0.0)
                    newd.append(dcar[h] + ex)
                    exj = [jnp.full((16,), jnp.max(
                        jnp.where(lanes == j, ex, 0.0)))
                           for j in range(16)]

                    def agg(si, c3):
                        off = h * D + si * 16
                        r = racc_v[pl.ds(off, 16)]
                        for j in range(16):
                            r = r + exj[j] * fu_v[j, pl.ds(off, 16)]
                        racc_v[pl.ds(off, 16)] = r
                        return c3

                    lax.fori_loop(0, D // 16, agg, 0)
                return tuple(newd)

            dfin = lax.fori_loop(0, nch, per_chunk, (zero16,) * H)
            invs = [1.0 / jnp.maximum(jnp.full((16,), jnp.sum(dfin[h])), 1e-9)
                    for h in range(H)]
            if residual_mean:
                pltpu.sync_copy(hres_hbm.at[v], hrow_v)

                def fl(si, c4):
                    acc = zero16
                    for h in range(H):
                        off = h * D + si * 16
                        acc = acc + racc_v[pl.ds(off, 16)] * invs[h] \
                            + hrow_v[pl.ds(off, 16)]
                    stg_v[pl.ds(si * 16, 16)] = acc * (1.0 / H)
                    return c4
            else:
                def fl(si, c4):
                    for h in range(H):
                        off = h * D + si * 16
                        r = racc_v[pl.ds(off, 16)] * invs[h]
                        stg_v[pl.ds(off, 16)] = jnp.maximum(r, 0.0)
                    return c4

            lax.fori_loop(0, D // 16, fl, 0)
            pltpu.sync_copy(stg_v, out_hbm.at[v])
            return carry

        lax.fori_loop(v0, v1, per_dst, 0)

    return edge_kernel


def _edge_setup(src, dst):
    order = jnp.argsort(dst)
    src_s = src[order].astype(jnp.int32)
    dst_s = dst[order]
    rowptr = jnp.searchsorted(dst_s, jnp.arange(N + 1, dtype=jnp.int32)
                              ).astype(jnp.int32)
    targets = (jnp.arange(NW + 1, dtype=jnp.int32) * E) // NW
    vstart = jnp.searchsorted(rowptr, targets).astype(jnp.int32)
    vstart = jnp.minimum(vstart, N).at[NW].set(N).at[0].set(0)
    srcs_pad = jnp.pad(src_s, (0, EPAD - E))
    rowptr_pad = jnp.pad(rowptr, (0, RPPAD - (N + 1)))
    vstart_pad = jnp.pad(vstart, (0, VSPAD - (NW + 1)))
    return srcs_pad, rowptr_pad, vstart_pad


def kernel(x, edge_index, W1, attn1, W2, attn2):
    src = edge_index[0]
    dst = edge_index[1]
    srcs_pad, rowptr_pad, vstart_pad = _edge_setup(src, dst)

    f1, g1, gm1 = _matmul_g(x, W1, attn1)
    gm1b = jnp.broadcast_to(gm1.reshape(1), (16,))
    h = _make_edge_kernel(False)(
        f1, g1.reshape(-1), gm1b, srcs_pad, rowptr_pad, vstart_pad,
        attn1.reshape(-1), f1)

    f2, g2, gm2 = _matmul_g(h, W2, attn2)
    gm2b = jnp.broadcast_to(gm2.reshape(1), (16,))
    out = _make_edge_kernel(True)(
        f2, g2.reshape(-1), gm2b, srcs_pad, rowptr_pad, vstart_pad,
        attn2.reshape(-1), h)
    return out


# speedup vs baseline: 6.5206x; 1.7353x over previous
import functools

import jax
import jax.numpy as jnp
from jax import lax
from jax.experimental import pallas as pl
from jax.experimental.pallas import tpu as pltpu
from jax.experimental.pallas import tpu_sc as plsc

N = 10000
E = 160000
H = 4
D = 256
HD = H * D
NEG_SLOPE = 0.2
NW = 32
EPAD = E + 32
RPPAD = 10008
VSPAD = 48

_M_TILE = 400


def _mm_body(h_ref, w_ref, attn_ref, f_ref, g_ref, gmax_ref):
    i = pl.program_id(0)
    f = jnp.dot(h_ref[...], w_ref[...], preferred_element_type=jnp.float32,
                precision=lax.Precision.HIGHEST)
    f_ref[...] = f
    a = attn_ref[...].reshape(1, -1)
    af = f * a
    lr = jnp.maximum(af, NEG_SLOPE * af)
    g = lr.reshape(_M_TILE, H, -1).sum(-1)
    g_ref[...] = g
    tile_max = jnp.max(g)

    @pl.when(i == 0)
    def _init():
        gmax_ref[0, 0] = tile_max

    @pl.when(i > 0)
    def _acc():
        gmax_ref[0, 0] = jnp.maximum(gmax_ref[0, 0], tile_max)


def _matmul_g(h, W, attn):
    n, k = h.shape
    hd = W.shape[1]
    f, g, gmax = pl.pallas_call(
        _mm_body,
        grid=(n // _M_TILE,),
        in_specs=[
            pl.BlockSpec((_M_TILE, k), lambda i: (i, 0)),
            pl.BlockSpec((k, hd), lambda i: (0, 0)),
            pl.BlockSpec((H, hd // H), lambda i: (0, 0)),
        ],
        out_specs=[
            pl.BlockSpec((_M_TILE, hd), lambda i: (i, 0)),
            pl.BlockSpec((_M_TILE, H), lambda i: (i, 0)),
            pl.BlockSpec(memory_space=pltpu.SMEM),
        ],
        out_shape=[
            jax.ShapeDtypeStruct((n, hd), jnp.float32),
            jax.ShapeDtypeStruct((n, H), jnp.float32),
            jax.ShapeDtypeStruct((1, 1), jnp.float32),
        ],
    )(h, W, attn)
    return f, g, gmax


def _make_edge_kernel(residual_mean):
    out_dim = D if residual_mean else HD
    mesh = plsc.VectorSubcoreMesh(core_axis_name="c", subcore_axis_name="s")
    scratch = [
        pltpu.VMEM((N * H,), jnp.float32),
        pltpu.VMEM((RPPAD,), jnp.int32),
        pltpu.VMEM((VSPAD,), jnp.int32),
        pltpu.VMEM((16,), jnp.float32),
        pltpu.VMEM((HD,), jnp.float32),
        pltpu.VMEM((HD,), jnp.float32),
        pltpu.VMEM((16, HD), jnp.float32),
        pltpu.VMEM((HD,), jnp.float32),
        pltpu.VMEM((out_dim,), jnp.float32),
        pltpu.VMEM((HD,), jnp.float32),
        pltpu.VMEM((16,), jnp.int32),
        pltpu.VMEM((32,), jnp.int32),
        pltpu.SemaphoreType.DMA,
    ]

    @functools.partial(
        pl.kernel,
        out_type=jax.ShapeDtypeStruct((N, out_dim), jnp.float32),
        mesh=mesh,
        scratch_types=scratch,
        compiler_params=pltpu.CompilerParams(needs_layout_passes=False),
    )
    def edge_kernel(f_hbm, g_hbm, gmax_hbm, srcs_hbm, rp_hbm, vs_hbm, attn_hbm,
                    hres_hbm, out_hbm, g_v, rp_v, vs_v, gmax_v, attn_v, fd_v,
                    fu_v, racc_v, stg_v, hrow_v, idx_v, sst_v, sem):
        cc = lax.axis_index("c")
        sc = lax.axis_index("s")
        wid = sc * 2 + cc
        pltpu.sync_copy(g_hbm, g_v)
        pltpu.sync_copy(rp_hbm, rp_v)
        pltpu.sync_copy(vs_hbm, vs_v)
        pltpu.sync_copy(gmax_hbm, gmax_v)
        pltpu.sync_copy(attn_hbm, attn_v)
        lanes = lax.iota(jnp.int32, 16)
        zero16 = jnp.zeros((16,), jnp.float32)
        gmaxv = gmax_v[...]

        def spl(x):
            return jnp.full((16,), x, jnp.int32)

        def ext(ref, i):
            return jnp.max(plsc.load_gather(ref, [spl(i)]))

        v0 = ext(vs_v, wid)
        v1 = ext(vs_v, wid + 1)

        def per_dst(v, carry):
            lo = ext(rp_v, v)
            hi = ext(rp_v, v + 1)
            pltpu.sync_copy(f_hbm.at[v], fd_v)

            def zr(i, c2):
                racc_v[pl.ds(i * 16, 16)] = zero16
                return c2

            lax.fori_loop(0, HD // 16, zr, 0)
            nch = (hi - lo + 15) // 16

            def per_chunk(ci, dcar):
                base = lo + ci * 16
                k = hi - base
                off8 = pl.multiple_of((base // 8) * 8, 8)
                pltpu.sync_copy(srcs_hbm.at[pl.ds(off8, 32)], sst_v)
                pos = (base - off8) + jnp.minimum(lanes, k - 1)
                idx_v[...] = plsc.load_gather(sst_v, [pos])
                pltpu.async_copy(f_hbm.at[idx_v], fu_v, sem).wait()
                mask = lanes < k
                c6 = 0.5 * (1 + NEG_SLOPE)
                c4 = 0.5 * (1 - NEG_SLOPE)
                newd = []
                for h in range(H):
                    def lg(s16, accs_t):
                        dbase = h * D + s16 * 16
                        fdv = fd_v[pl.ds(dbase, 16)]
                        av = attn_v[pl.ds(dbase, 16)]
                        a6 = c6 * av
                        a4 = c4 * av
                        out = []
                        for j in range(16):
                            z = fu_v[j, pl.ds(dbase, 16)] + fdv
                            out.append(accs_t[j] + a6 * z + a4 * jnp.abs(z))
                        return tuple(out)

                    accs = lax.fori_loop(0, D // 16, lg, (zero16,) * 16)
                    logitv = zero16
                    for j in range(16):
                        logitv = jnp.where(lanes == j,
                                           jnp.full((16,), jnp.sum(accs[j])),
                                           logitv)
                    mh = plsc.load_gather(g_v, [spl(v * H + h)]) + gmaxv
                    ex = jnp.exp(logitv - mh)
                    ex = jnp.where(mask, ex, 0.0)
                    newd.append(dcar[h] + ex)
                    exj = [jnp.full((16,), jnp.max(
                        jnp.where(lanes == j, ex, 0.0)))
                           for j in range(16)]

                    def agg(si, c3):
                        off = h * D + si * 16
                        r = racc_v[pl.ds(off, 16)]
                        for j in range(16):
                            r = r + exj[j] * fu_v[j, pl.ds(off, 16)]
                        racc_v[pl.ds(off, 16)] = r
                        return c3

                    lax.fori_loop(0, D // 16, agg, 0)
                return tuple(newd)

            dfin = lax.fori_loop(0, nch, per_chunk, (zero16,) * H)
            invs = [1.0 / jnp.maximum(jnp.full((16,), jnp.sum(dfin[h])), 1e-9)
                    for h in range(H)]
            if residual_mean:
                pltpu.sync_copy(hres_hbm.at[v], hrow_v)

                def fl(si, c4):
                    acc = zero16
                    for h in range(H):
                        off = h * D + si * 16
                        acc = acc + racc_v[pl.ds(off, 16)] * invs[h] \
                            + hrow_v[pl.ds(off, 16)]
                    stg_v[pl.ds(si * 16, 16)] = acc * (1.0 / H)
                    return c4
            else:
                def fl(si, c4):
                    for h in range(H):
                        off = h * D + si * 16
                        r = racc_v[pl.ds(off, 16)] * invs[h]
                        stg_v[pl.ds(off, 16)] = jnp.maximum(r, 0.0)
                    return c4

            lax.fori_loop(0, D // 16, fl, 0)
            pltpu.sync_copy(stg_v, out_hbm.at[v])
            return carry

        lax.fori_loop(v0, v1, per_dst, 0)

    return edge_kernel


def _edge_setup(src, dst):
    packed = jnp.sort(dst * 16384 + src)
    src_s = (packed & 16383).astype(jnp.int32)
    rowptr = jnp.searchsorted(
        packed, jnp.arange(N + 1, dtype=jnp.int32) * 16384).astype(jnp.int32)
    targets = (jnp.arange(NW + 1, dtype=jnp.int32) * E) // NW
    vstart = jnp.searchsorted(rowptr, targets).astype(jnp.int32)
    vstart = jnp.minimum(vstart, N).at[NW].set(N).at[0].set(0)
    srcs_pad = jnp.pad(src_s, (0, EPAD - E))
    rowptr_pad = jnp.pad(rowptr, (0, RPPAD - (N + 1)))
    vstart_pad = jnp.pad(vstart, (0, VSPAD - (NW + 1)))
    return srcs_pad, rowptr_pad, vstart_pad


def kernel(x, edge_index, W1, attn1, W2, attn2):
    src = edge_index[0]
    dst = edge_index[1]
    srcs_pad, rowptr_pad, vstart_pad = _edge_setup(src, dst)

    f1, g1, gm1 = _matmul_g(x, W1, attn1)
    gm1b = jnp.broadcast_to(gm1.reshape(1), (16,))
    h = _make_edge_kernel(False)(
        f1, g1.reshape(-1), gm1b, srcs_pad, rowptr_pad, vstart_pad,
        attn1.reshape(-1), f1)

    f2, g2, gm2 = _matmul_g(h, W2, attn2)
    gm2b = jnp.broadcast_to(gm2.reshape(1), (16,))
    out = _make_edge_kernel(True)(
        f2, g2.reshape(-1), gm2b, srcs_pad, rowptr_pad, vstart_pad,
        attn2.reshape(-1), h)
    return out
